# Initial kernel scaffold; baseline (speedup 1.0000x reference)
import jax
import jax.numpy as jnp
from jax.experimental import pallas as pl


def kernel(cls_scores, bbox_preds, gt_bboxes, gt_labels):
    raise NotImplementedError("write your pallas kernel here")


# speedup vs baseline: 41.2200x; 41.2200x over previous
import jax
import jax.numpy as jnp
from jax.experimental import pallas as pl

_B, _N, _C, _G = 8, 5000, 80, 60
_TOPK = 8
_EPS = 1e-12


def _body(clsT_ref, bboxT_ref, gtb_ref, gtl_ref, out_ref):
    csT = clsT_ref[0]
    bp = bboxT_ref[0]
    gb = gtb_ref[0]
    gl = gtl_ref[0]

    px1 = bp[0:1, :]
    py1 = bp[1:2, :]
    px2 = bp[2:3, :]
    py2 = bp[3:4, :]
    area_p = (px2 - px1) * (py2 - py1)
    gx1 = gb[:, 0:1]
    gy1 = gb[:, 1:2]
    gx2 = gb[:, 2:3]
    gy2 = gb[:, 3:4]
    area_g = (gx2 - gx1) * (gy2 - gy1)

    w = jnp.clip(jnp.minimum(px2, gx2) - jnp.maximum(px1, gx1), 0.0, None)
    h = jnp.clip(jnp.minimum(py2, gy2) - jnp.maximum(py1, gy1), 0.0, None)
    inter = w * h
    union = area_p + area_g - inter
    iou = inter / jnp.maximum(union, 1e-7)

    iota_c = jax.lax.broadcasted_iota(jnp.int32, (_G, _C), 1)
    onehot = (gl == iota_c).astype(jnp.float32)
    cs_at = jax.lax.dot_general(onehot, csT, (((1,), (0,)), ((), ())),
                                preferred_element_type=jnp.float32)
    i2 = iou * iou
    metric = cs_at * (i2 * i2 * i2)

    iota_n = jax.lax.broadcasted_iota(jnp.int32, (_G, _N), 1)
    work = metric
    cand = jnp.zeros((_G, _N), dtype=jnp.bool_)
    for _ in range(_TOPK):
        m = jnp.max(work, axis=1, keepdims=True)
        idx = jnp.min(jnp.where(work == m, iota_n, _N), axis=1, keepdims=True)
        hit = iota_n == idx
        cand = jnp.logical_or(cand, hit)
        work = jnp.where(hit, -1.0, work)

    cand_metric = jnp.where(cand, metric, -1e9)
    best = jnp.max(cand_metric, axis=0, keepdims=True)
    iota_g = jax.lax.broadcasted_iota(jnp.int32, (_G, _N), 0)
    assigned = jnp.min(jnp.where(cand_metric == best, iota_g, _G),
                       axis=0, keepdims=True)
    is_pos = jnp.max(jnp.where(cand, 1.0, 0.0), axis=0, keepdims=True) > 0.0
    sel = jnp.logical_and(iota_g == assigned, is_pos)

    assign_metric = jnp.where(is_pos, best, 0.0)
    assign_iou = jnp.sum(jnp.where(sel, iou, 0.0), axis=0, keepdims=True)

    max_metric_g = jnp.max(jnp.where(sel, assign_metric, 0.0), axis=1, keepdims=True)
    max_iou_g = jnp.max(jnp.where(sel, assign_iou, 0.0), axis=1, keepdims=True)
    mm_at = jnp.sum(jnp.where(sel, max_metric_g, 0.0), axis=0, keepdims=True)
    mi_at = jnp.sum(jnp.where(sel, max_iou_g, 0.0), axis=0, keepdims=True)
    norm_metric = jnp.where(is_pos, assign_metric / (mm_at + 1e-7) * mi_at, 0.0)

    gl_f = gl.astype(jnp.float32)
    lab_row = jnp.sum(jnp.where(sel, gl_f, 0.0), axis=0, keepdims=True)
    lab_i = lab_row.astype(jnp.int32)

    p = jnp.clip(csT, _EPS, 1.0 - _EPS)
    neg = -jnp.log(1.0 - p) * p * p
    neg_sum = jnp.sum(neg)
    iota_cc = jax.lax.broadcasted_iota(jnp.int32, (_C, _N), 0)
    labhit = iota_cc == lab_i
    p_pos = jnp.sum(jnp.where(labhit, p, 0.0), axis=0, keepdims=True)
    neg_at = jnp.sum(jnp.where(labhit, neg, 0.0), axis=0, keepdims=True)
    score = norm_metric
    bce = -(score * jnp.log(p_pos) + (1.0 - score) * jnp.log(1.0 - p_pos))
    d = jnp.abs(score - p_pos)
    pos_loss = bce * d * d
    loss_cls = neg_sum + jnp.sum(jnp.where(is_pos, pos_loss - neg_at, 0.0))

    tx1 = jnp.sum(jnp.where(sel, gx1, 0.0), axis=0, keepdims=True)
    ty1 = jnp.sum(jnp.where(sel, gy1, 0.0), axis=0, keepdims=True)
    tx2 = jnp.sum(jnp.where(sel, gx2, 0.0), axis=0, keepdims=True)
    ty2 = jnp.sum(jnp.where(sel, gy2, 0.0), axis=0, keepdims=True)
    iw = jnp.clip(jnp.minimum(px2, tx2) - jnp.maximum(px1, tx1), 0.0, None)
    ih = jnp.clip(jnp.minimum(py2, ty2) - jnp.maximum(py1, ty1), 0.0, None)
    inter2 = iw * ih
    at = (tx2 - tx1) * (ty2 - ty1)
    union2 = area_p + at - inter2
    iou2 = inter2 / jnp.maximum(union2, 1e-7)
    ew = jnp.clip(jnp.maximum(px2, tx2) - jnp.minimum(px1, tx1), 0.0, None)
    eh = jnp.clip(jnp.maximum(py2, ty2) - jnp.minimum(py1, ty1), 0.0, None)
    enclose = ew * eh
    giou = iou2 - (enclose - union2) / jnp.maximum(enclose, 1e-7)
    loss_bbox = jnp.sum((1.0 - giou) * norm_metric) * 2.0
    af = jnp.sum(norm_metric)

    lane = jax.lax.broadcasted_iota(jnp.int32, (1, 128), 1)
    row = (jnp.where(lane == 0, loss_cls, 0.0)
           + jnp.where(lane == 1, loss_bbox, 0.0)
           + jnp.where(lane == 2, af, 0.0))
    out_ref[0] = row


def _aux_loss(cls_scores, bbox_preds, gt_bboxes, gt_labels, interpret=False):
    clsT = jnp.transpose(cls_scores, (0, 2, 1))
    bboxT = jnp.transpose(bbox_preds, (0, 2, 1))
    gl3 = gt_labels.astype(jnp.int32).reshape(_B, _G, 1)
    out = pl.pallas_call(
        _body,
        grid=(_B,),
        in_specs=[
            pl.BlockSpec((1, _C, _N), lambda b: (b, 0, 0)),
            pl.BlockSpec((1, 4, _N), lambda b: (b, 0, 0)),
            pl.BlockSpec((1, _G, 4), lambda b: (b, 0, 0)),
            pl.BlockSpec((1, _G, 1), lambda b: (b, 0, 0)),
        ],
        out_specs=pl.BlockSpec((1, 1, 128), lambda b: (b, 0, 0)),
        out_shape=jax.ShapeDtypeStruct((_B, 1, 128), jnp.float32),
        interpret=interpret,
    )(clsT, bboxT, gt_bboxes, gl3)
    lc = out[:, 0, 0]
    lb = out[:, 0, 1]
    af = out[:, 0, 2]
    cls_avg = jnp.clip(jnp.sum(af), 1.0, None)
    bbox_avg = jnp.clip(jnp.sum(af), 1.0, None)
    return jnp.stack([lc / cls_avg, lb / bbox_avg])


@jax.jit
def kernel(cls_scores, bbox_preds, gt_bboxes, gt_labels):
    return _aux_loss(cls_scores, bbox_preds, gt_bboxes, gt_labels)


# speedup vs baseline: 53.7716x; 1.3045x over previous
import jax
import jax.numpy as jnp
from jax.experimental import pallas as pl

_B, _N, _C, _G = 8, 5000, 80, 60
_TOPK = 8
_EPS = 1e-12


def _body(clsT_ref, bboxT_ref, gtb_ref, gtl_ref, out_ref):
    csT = clsT_ref[0]
    bp = bboxT_ref[0]
    gb = gtb_ref[0]
    gl = gtl_ref[0]

    px1 = bp[0:1, :]
    py1 = bp[1:2, :]
    px2 = bp[2:3, :]
    py2 = bp[3:4, :]
    area_p = (px2 - px1) * (py2 - py1)
    gx1 = gb[:, 0:1]
    gy1 = gb[:, 1:2]
    gx2 = gb[:, 2:3]
    gy2 = gb[:, 3:4]
    area_g = (gx2 - gx1) * (gy2 - gy1)

    w = jnp.clip(jnp.minimum(px2, gx2) - jnp.maximum(px1, gx1), 0.0, None)
    h = jnp.clip(jnp.minimum(py2, gy2) - jnp.maximum(py1, gy1), 0.0, None)
    inter = w * h
    union = area_p + area_g - inter
    iou = inter / jnp.maximum(union, 1e-7)

    iota_c = jax.lax.broadcasted_iota(jnp.int32, (_G, _C), 1)
    onehot = (gl == iota_c).astype(jnp.float32)
    cs_at = jax.lax.dot_general(onehot, csT, (((1,), (0,)), ((), ())),
                                preferred_element_type=jnp.float32)
    i2 = iou * iou
    metric = cs_at * (i2 * i2 * i2)

    iota_n = jax.lax.broadcasted_iota(jnp.int32, (_G, _N), 1)
    work = metric
    for _ in range(_TOPK):
        m = jnp.max(work, axis=1, keepdims=True)
        idx = jnp.min(jnp.where(work == m, iota_n, _N), axis=1, keepdims=True)
        work = jnp.where(iota_n == idx, -1.0, work)
    cand = work < 0.0

    cand_metric = jnp.where(cand, metric, -1e9)
    best = jnp.max(cand_metric, axis=0, keepdims=True)
    iota_g = jax.lax.broadcasted_iota(jnp.int32, (_G, _N), 0)
    assigned = jnp.min(jnp.where(cand_metric == best, iota_g, _G),
                       axis=0, keepdims=True)
    is_pos = best >= 0.0
    sel0 = iota_g == assigned
    sel_f = sel0.astype(jnp.float32)

    assign_metric = jnp.where(is_pos, best, 0.0)
    assign_iou = jnp.where(
        is_pos, jnp.sum(jnp.where(sel0, iou, 0.0), axis=0, keepdims=True), 0.0)

    max_metric_g = jnp.max(jnp.where(sel0, assign_metric, 0.0), axis=1, keepdims=True)
    max_iou_g = jnp.max(jnp.where(sel0, assign_iou, 0.0), axis=1, keepdims=True)

    gl_f = gl.astype(jnp.float32)
    a_cols = jnp.concatenate(
        [gx1, gy1, gx2, gy2, gl_f, max_metric_g, max_iou_g], axis=1)
    r = jax.lax.dot_general(a_cols, sel_f, (((0,), (0,)), ((), ())),
                            preferred_element_type=jnp.float32)
    tx1 = r[0:1, :]
    ty1 = r[1:2, :]
    tx2 = r[2:3, :]
    ty2 = r[3:4, :]
    mm_at = r[5:6, :]
    mi_at = r[6:7, :]
    norm_metric = jnp.where(is_pos, assign_metric / (mm_at + 1e-7) * mi_at, 0.0)
    lab_i = jnp.where(is_pos, r[4:5, :], 0.0).astype(jnp.int32)

    p = jnp.clip(csT, _EPS, 1.0 - _EPS)
    neg = -jnp.log(1.0 - p) * p * p
    neg_sum = jnp.sum(neg)
    iota_cc = jax.lax.broadcasted_iota(jnp.int32, (_C, _N), 0)
    labhit = iota_cc == lab_i
    p_pos = jnp.sum(jnp.where(labhit, p, 0.0), axis=0, keepdims=True)
    neg_at = jnp.sum(jnp.where(labhit, neg, 0.0), axis=0, keepdims=True)
    score = norm_metric
    bce = -(score * jnp.log(p_pos) + (1.0 - score) * jnp.log(1.0 - p_pos))
    d = jnp.abs(score - p_pos)
    pos_loss = bce * d * d
    loss_cls = neg_sum + jnp.sum(jnp.where(is_pos, pos_loss - neg_at, 0.0))

    iw = jnp.clip(jnp.minimum(px2, tx2) - jnp.maximum(px1, tx1), 0.0, None)
    ih = jnp.clip(jnp.minimum(py2, ty2) - jnp.maximum(py1, ty1), 0.0, None)
    inter2 = iw * ih
    at = (tx2 - tx1) * (ty2 - ty1)
    union2 = area_p + at - inter2
    iou2 = inter2 / jnp.maximum(union2, 1e-7)
    ew = jnp.clip(jnp.maximum(px2, tx2) - jnp.minimum(px1, tx1), 0.0, None)
    eh = jnp.clip(jnp.maximum(py2, ty2) - jnp.minimum(py1, ty1), 0.0, None)
    enclose = ew * eh
    giou = iou2 - (enclose - union2) / jnp.maximum(enclose, 1e-7)
    loss_bbox = jnp.sum((1.0 - giou) * norm_metric) * 2.0
    af = jnp.sum(norm_metric)

    lane = jax.lax.broadcasted_iota(jnp.int32, (1, 128), 1)
    row = (jnp.where(lane == 0, loss_cls, 0.0)
           + jnp.where(lane == 1, loss_bbox, 0.0)
           + jnp.where(lane == 2, af, 0.0))
    out_ref[0] = row


def _aux_loss(cls_scores, bbox_preds, gt_bboxes, gt_labels, interpret=False):
    clsT = jnp.transpose(cls_scores, (0, 2, 1))
    bboxT = jnp.transpose(bbox_preds, (0, 2, 1))
    gl3 = gt_labels.astype(jnp.int32).reshape(_B, _G, 1)
    out = pl.pallas_call(
        _body,
        grid=(_B,),
        in_specs=[
            pl.BlockSpec((1, _C, _N), lambda b: (b, 0, 0)),
            pl.BlockSpec((1, 4, _N), lambda b: (b, 0, 0)),
            pl.BlockSpec((1, _G, 4), lambda b: (b, 0, 0)),
            pl.BlockSpec((1, _G, 1), lambda b: (b, 0, 0)),
        ],
        out_specs=pl.BlockSpec((1, 1, 128), lambda b: (b, 0, 0)),
        out_shape=jax.ShapeDtypeStruct((_B, 1, 128), jnp.float32),
        interpret=interpret,
    )(clsT, bboxT, gt_bboxes, gl3)
    lc = out[:, 0, 0]
    lb = out[:, 0, 1]
    af = out[:, 0, 2]
    cls_avg = jnp.clip(jnp.sum(af), 1.0, None)
    bbox_avg = jnp.clip(jnp.sum(af), 1.0, None)
    return jnp.stack([lc / cls_avg, lb / bbox_avg])


@jax.jit
def kernel(cls_scores, bbox_preds, gt_bboxes, gt_labels):
    return _aux_loss(cls_scores, bbox_preds, gt_bboxes, gt_labels)


# speedup vs baseline: 73.7082x; 1.3708x over previous
import jax
import jax.numpy as jnp
from jax.experimental import pallas as pl

_B, _N, _C, _G = 8, 5000, 80, 60
_TOPK = 8
_EPS = 1e-12


def _body(clsT_ref, bboxT_ref, gtb_ref, gtl_ref, out_ref):
    csT = clsT_ref[0]
    bp = bboxT_ref[0]
    gb = gtb_ref[0]
    gl = gtl_ref[0]

    px1 = bp[0:1, :]
    py1 = bp[1:2, :]
    px2 = bp[2:3, :]
    py2 = bp[3:4, :]
    area_p = (px2 - px1) * (py2 - py1)
    gx1 = gb[:, 0:1]
    gy1 = gb[:, 1:2]
    gx2 = gb[:, 2:3]
    gy2 = gb[:, 3:4]
    area_g = (gx2 - gx1) * (gy2 - gy1)

    w = jnp.clip(jnp.minimum(px2, gx2) - jnp.maximum(px1, gx1), 0.0, None)
    h = jnp.clip(jnp.minimum(py2, gy2) - jnp.maximum(py1, gy1), 0.0, None)
    inter = w * h
    union = area_p + area_g - inter
    iou = inter / jnp.maximum(union, 1e-7)

    iota_c = jax.lax.broadcasted_iota(jnp.int32, (_G, _C), 1)
    onehot = (gl == iota_c).astype(jnp.float32)
    cs_at = jax.lax.dot_general(onehot, csT, (((1,), (0,)), ((), ())),
                                preferred_element_type=jnp.float32)
    i2 = iou * iou
    metric = cs_at * (i2 * i2 * i2)

    iota_n = jax.lax.broadcasted_iota(jnp.int32, (_G, _N), 1)
    tiny = jnp.float32(2.0 ** -126)
    zkey = (iota_n + 1).astype(jnp.float32) * (-tiny)
    work = jnp.where(metric > 0.0, metric, zkey)
    for _ in range(_TOPK):
        m = jnp.max(work, axis=1, keepdims=True)
        work = jnp.where(work == m, -1e30, work)
    cand = work < -1e29

    cand_metric = jnp.where(cand, metric, -1e9)
    best = jnp.max(cand_metric, axis=0, keepdims=True)
    iota_g = jax.lax.broadcasted_iota(jnp.int32, (_G, _N), 0)
    assigned = jnp.min(jnp.where(cand_metric == best, iota_g, _G),
                       axis=0, keepdims=True)
    is_pos = best >= 0.0
    sel0 = iota_g == assigned
    sel_f = sel0.astype(jnp.float32)

    assign_metric = jnp.where(is_pos, best, 0.0)
    assign_iou = jnp.where(
        is_pos, jnp.sum(jnp.where(sel0, iou, 0.0), axis=0, keepdims=True), 0.0)

    max_metric_g = jnp.max(jnp.where(sel0, assign_metric, 0.0), axis=1, keepdims=True)
    max_iou_g = jnp.max(jnp.where(sel0, assign_iou, 0.0), axis=1, keepdims=True)

    gl_f = gl.astype(jnp.float32)
    a_cols = jnp.concatenate(
        [gx1, gy1, gx2, gy2, gl_f, max_metric_g, max_iou_g], axis=1)
    r = jax.lax.dot_general(a_cols, sel_f, (((0,), (0,)), ((), ())),
                            preferred_element_type=jnp.float32)
    tx1 = r[0:1, :]
    ty1 = r[1:2, :]
    tx2 = r[2:3, :]
    ty2 = r[3:4, :]
    mm_at = r[5:6, :]
    mi_at = r[6:7, :]
    norm_metric = jnp.where(is_pos, assign_metric / (mm_at + 1e-7) * mi_at, 0.0)
    lab_i = jnp.where(is_pos, r[4:5, :], 0.0).astype(jnp.int32)

    p = jnp.clip(csT, _EPS, 1.0 - _EPS)
    neg = -jnp.log(1.0 - p) * p * p
    neg_sum = jnp.sum(neg)
    iota_cc = jax.lax.broadcasted_iota(jnp.int32, (_C, _N), 0)
    labhit = iota_cc == lab_i
    p_pos = jnp.sum(jnp.where(labhit, p, 0.0), axis=0, keepdims=True)
    neg_at = jnp.sum(jnp.where(labhit, neg, 0.0), axis=0, keepdims=True)
    score = norm_metric
    bce = -(score * jnp.log(p_pos) + (1.0 - score) * jnp.log(1.0 - p_pos))
    d = jnp.abs(score - p_pos)
    pos_loss = bce * d * d
    loss_cls = neg_sum + jnp.sum(jnp.where(is_pos, pos_loss - neg_at, 0.0))

    iw = jnp.clip(jnp.minimum(px2, tx2) - jnp.maximum(px1, tx1), 0.0, None)
    ih = jnp.clip(jnp.minimum(py2, ty2) - jnp.maximum(py1, ty1), 0.0, None)
    inter2 = iw * ih
    at = (tx2 - tx1) * (ty2 - ty1)
    union2 = area_p + at - inter2
    iou2 = inter2 / jnp.maximum(union2, 1e-7)
    ew = jnp.clip(jnp.maximum(px2, tx2) - jnp.minimum(px1, tx1), 0.0, None)
    eh = jnp.clip(jnp.maximum(py2, ty2) - jnp.minimum(py1, ty1), 0.0, None)
    enclose = ew * eh
    giou = iou2 - (enclose - union2) / jnp.maximum(enclose, 1e-7)
    loss_bbox = jnp.sum((1.0 - giou) * norm_metric) * 2.0
    af = jnp.sum(norm_metric)

    lane = jax.lax.broadcasted_iota(jnp.int32, (1, 128), 1)
    row = (jnp.where(lane == 0, loss_cls, 0.0)
           + jnp.where(lane == 1, loss_bbox, 0.0)
           + jnp.where(lane == 2, af, 0.0))
    out_ref[0] = row


def _aux_loss(cls_scores, bbox_preds, gt_bboxes, gt_labels, interpret=False):
    clsT = jnp.transpose(cls_scores, (0, 2, 1))
    bboxT = jnp.transpose(bbox_preds, (0, 2, 1))
    gl3 = gt_labels.astype(jnp.int32).reshape(_B, _G, 1)
    out = pl.pallas_call(
        _body,
        grid=(_B,),
        in_specs=[
            pl.BlockSpec((1, _C, _N), lambda b: (b, 0, 0)),
            pl.BlockSpec((1, 4, _N), lambda b: (b, 0, 0)),
            pl.BlockSpec((1, _G, 4), lambda b: (b, 0, 0)),
            pl.BlockSpec((1, _G, 1), lambda b: (b, 0, 0)),
        ],
        out_specs=pl.BlockSpec((1, 1, 128), lambda b: (b, 0, 0)),
        out_shape=jax.ShapeDtypeStruct((_B, 1, 128), jnp.float32),
        interpret=interpret,
    )(clsT, bboxT, gt_bboxes, gl3)
    lc = out[:, 0, 0]
    lb = out[:, 0, 1]
    af = out[:, 0, 2]
    cls_avg = jnp.clip(jnp.sum(af), 1.0, None)
    bbox_avg = jnp.clip(jnp.sum(af), 1.0, None)
    return jnp.stack([lc / cls_avg, lb / bbox_avg])


@jax.jit
def kernel(cls_scores, bbox_preds, gt_bboxes, gt_labels):
    return _aux_loss(cls_scores, bbox_preds, gt_bboxes, gt_labels)


# speedup vs baseline: 75.2458x; 1.0209x over previous
import jax
import jax.numpy as jnp
from jax.experimental import pallas as pl

_B, _N, _C, _G = 8, 5000, 80, 60
_TOPK = 8
_EPS = 1e-12


def _body(clsT_ref, bboxT_ref, gtb_ref, gtl_ref, zkey_ref, gfill_ref, out_ref):
    csT = clsT_ref[0]
    bp = bboxT_ref[0]
    gb = gtb_ref[0]
    gl = gtl_ref[0]

    px1 = bp[0:1, :]
    py1 = bp[1:2, :]
    px2 = bp[2:3, :]
    py2 = bp[3:4, :]
    area_p = (px2 - px1) * (py2 - py1)
    gx1 = gb[:, 0:1]
    gy1 = gb[:, 1:2]
    gx2 = gb[:, 2:3]
    gy2 = gb[:, 3:4]
    area_g = (gx2 - gx1) * (gy2 - gy1)

    w = jnp.clip(jnp.minimum(px2, gx2) - jnp.maximum(px1, gx1), 0.0, None)
    h = jnp.clip(jnp.minimum(py2, gy2) - jnp.maximum(py1, gy1), 0.0, None)
    inter = w * h
    union = area_p + area_g - inter
    iou = inter / jnp.maximum(union, 1e-7)

    iota_c = jax.lax.broadcasted_iota(jnp.int32, (_G, _C), 1)
    onehot = (gl == iota_c).astype(jnp.float32)
    cs_at = jax.lax.dot_general(onehot, csT, (((1,), (0,)), ((), ())),
                                preferred_element_type=jnp.float32)
    i2 = iou * iou
    metric = cs_at * (i2 * i2 * i2)

    work = jnp.where(metric > 0.0, metric, zkey_ref[0][0:1, :])
    for _ in range(_TOPK):
        m = jnp.max(work, axis=1, keepdims=True)
        work = jnp.where(work == m, -1e30, work)
    cand = work < -1e29

    gfill = gfill_ref[0]
    cand_metric = jnp.where(cand, metric, gfill)
    best = jnp.max(cand_metric, axis=0, keepdims=True)
    is_pos = best >= 0.0
    sel0 = cand_metric == best
    sel_f = sel0.astype(jnp.float32)

    assign_metric = jnp.where(is_pos, best, 0.0)
    assign_iou = jnp.where(
        is_pos, jnp.sum(jnp.where(sel0, iou, 0.0), axis=0, keepdims=True), 0.0)

    max_metric_g = jnp.max(jnp.where(sel0, assign_metric, 0.0), axis=1, keepdims=True)
    max_iou_g = jnp.max(jnp.where(sel0, assign_iou, 0.0), axis=1, keepdims=True)

    gl_f = gl.astype(jnp.float32)
    a_cols = jnp.concatenate(
        [gx1, gy1, gx2, gy2, gl_f, max_metric_g, max_iou_g], axis=1)
    r = jax.lax.dot_general(a_cols, sel_f, (((0,), (0,)), ((), ())),
                            preferred_element_type=jnp.float32)
    tx1 = r[0:1, :]
    ty1 = r[1:2, :]
    tx2 = r[2:3, :]
    ty2 = r[3:4, :]
    mm_at = r[5:6, :]
    mi_at = r[6:7, :]
    norm_metric = jnp.where(is_pos, assign_metric / (mm_at + 1e-7) * mi_at, 0.0)
    lab_i = jnp.where(is_pos, r[4:5, :], 0.0).astype(jnp.int32)

    p = jnp.clip(csT, _EPS, 1.0 - _EPS)
    neg = -jnp.log(1.0 - p) * p * p
    neg_sum = jnp.sum(neg)
    iota_cc = jax.lax.broadcasted_iota(jnp.int32, (_C, _N), 0)
    labhit = iota_cc == lab_i
    p_pos = jnp.sum(jnp.where(labhit, p, 0.0), axis=0, keepdims=True)
    neg_at = jnp.sum(jnp.where(labhit, neg, 0.0), axis=0, keepdims=True)
    score = norm_metric
    bce = -(score * jnp.log(p_pos) + (1.0 - score) * jnp.log(1.0 - p_pos))
    d = jnp.abs(score - p_pos)
    pos_loss = bce * d * d
    loss_cls = neg_sum + jnp.sum(jnp.where(is_pos, pos_loss - neg_at, 0.0))

    t4 = r[0:4, :]
    mx = jnp.maximum(bp, t4)
    mn = jnp.minimum(bp, t4)
    iw = jnp.clip(mn[2:3, :] - mx[0:1, :], 0.0, None)
    ih = jnp.clip(mn[3:4, :] - mx[1:2, :], 0.0, None)
    inter2 = iw * ih
    at = (t4[2:3, :] - t4[0:1, :]) * (t4[3:4, :] - t4[1:2, :])
    union2 = area_p + at - inter2
    iou2 = inter2 / jnp.maximum(union2, 1e-7)
    ew = jnp.clip(mx[2:3, :] - mn[0:1, :], 0.0, None)
    eh = jnp.clip(mx[3:4, :] - mn[1:2, :], 0.0, None)
    enclose = ew * eh
    giou = iou2 - (enclose - union2) / jnp.maximum(enclose, 1e-7)
    loss_bbox = jnp.sum((1.0 - giou) * norm_metric) * 2.0
    af = jnp.sum(norm_metric)

    lane = jax.lax.broadcasted_iota(jnp.int32, (1, 128), 1)
    row = (jnp.where(lane == 0, loss_cls, 0.0)
           + jnp.where(lane == 1, loss_bbox, 0.0)
           + jnp.where(lane == 2, af, 0.0))
    out_ref[0] = row


def _aux_loss(cls_scores, bbox_preds, gt_bboxes, gt_labels, interpret=False):
    clsT = jnp.transpose(cls_scores, (0, 2, 1))
    bboxT = jnp.transpose(bbox_preds, (0, 2, 1))
    gl3 = gt_labels.astype(jnp.int32).reshape(_B, _G, 1)
    zkey = (-(2.0 ** -126)) * (jnp.arange(_N, dtype=jnp.float32) + 1.0)
    zkey = zkey.reshape(1, 1, _N)
    gfill = -(1e9 + 1024.0 * jnp.arange(_G, dtype=jnp.float32))
    gfill = gfill.reshape(1, _G, 1)
    out = pl.pallas_call(
        _body,
        grid=(_B,),
        in_specs=[
            pl.BlockSpec((1, _C, _N), lambda b: (b, 0, 0)),
            pl.BlockSpec((1, 4, _N), lambda b: (b, 0, 0)),
            pl.BlockSpec((1, _G, 4), lambda b: (b, 0, 0)),
            pl.BlockSpec((1, _G, 1), lambda b: (b, 0, 0)),
            pl.BlockSpec((1, 1, _N), lambda b: (0, 0, 0)),
            pl.BlockSpec((1, _G, 1), lambda b: (0, 0, 0)),
        ],
        out_specs=pl.BlockSpec((1, 1, 128), lambda b: (b, 0, 0)),
        out_shape=jax.ShapeDtypeStruct((_B, 1, 128), jnp.float32),
        interpret=interpret,
    )(clsT, bboxT, gt_bboxes, gl3, zkey, gfill)
    lc = out[:, 0, 0]
    lb = out[:, 0, 1]
    af = out[:, 0, 2]
    cls_avg = jnp.clip(jnp.sum(af), 1.0, None)
    bbox_avg = jnp.clip(jnp.sum(af), 1.0, None)
    return jnp.stack([lc / cls_avg, lb / bbox_avg])


@jax.jit
def kernel(cls_scores, bbox_preds, gt_bboxes, gt_labels):
    return _aux_loss(cls_scores, bbox_preds, gt_bboxes, gt_labels)
